# baseline (speedup 1.0000x reference)
import functools

import jax
import jax.numpy as jnp
from jax import lax
from jax.experimental import pallas as pl
from jax.experimental.pallas import tpu as pltpu
from jax.experimental.pallas import tpu_sc as plsc

SEQ = 200
DIM = 32
BATCH = 1024
NW = 32
ROWS_PER_W = BATCH // NW
ROWS_PER_CHUNK = 4
CHUNKS_PER_W = ROWS_PER_W // ROWS_PER_CHUNK
TOK_PER_CHUNK = ROWS_PER_CHUNK * SEQ
GATHER_LEN = 100
GATHERS_PER_CHUNK = TOK_PER_CHUNK // GATHER_LEN
LANES = 16


def _sc_body(idx_hbm, tok_hbm, pos_hbm, out_hbm, idx_v, pos_v, rows_v, sem):
    cid = lax.axis_index("c")
    sid = lax.axis_index("s")
    wid = sid * 2 + cid

    pltpu.sync_copy(idx_hbm.at[wid], idx_v)
    pltpu.sync_copy(pos_hbm, pos_v)

    for c in range(CHUNKS_PER_W):
        copies = []
        for j in range(GATHERS_PER_CHUNK):
            copies.append(pltpu.async_copy(
                tok_hbm.at[idx_v.at[c * GATHERS_PER_CHUNK + j]],
                rows_v.at[j // 2, pl.ds((j % 2) * GATHER_LEN, GATHER_LEN)],
                sem))
        for cp in copies:
            cp.wait()

        def add_l(l, _):
            p0 = pos_v[l, pl.ds(0, LANES)]
            p1 = pos_v[l, pl.ds(LANES, LANES)]
            for r in range(ROWS_PER_CHUNK):
                rows_v[r, l, pl.ds(0, LANES)] += p0
                rows_v[r, l, pl.ds(LANES, LANES)] += p1
            return 0

        lax.fori_loop(0, SEQ, add_l, 0)

        base = (wid * CHUNKS_PER_W + c) * ROWS_PER_CHUNK
        pltpu.sync_copy(rows_v, out_hbm.at[pl.ds(base, ROWS_PER_CHUNK)])


@jax.jit
def _run(idx, token_table, pos_table):
    mesh = plsc.VectorSubcoreMesh(core_axis_name="c", subcore_axis_name="s")
    return pl.kernel(
        _sc_body,
        out_type=jax.ShapeDtypeStruct((BATCH, SEQ, DIM), jnp.float32),
        mesh=mesh,
        scratch_types=[
            pltpu.VMEM((CHUNKS_PER_W * GATHERS_PER_CHUNK, GATHER_LEN),
                       jnp.int32),
            pltpu.VMEM((SEQ, DIM), jnp.float32),
            pltpu.VMEM((ROWS_PER_CHUNK, SEQ, DIM), jnp.float32),
            pltpu.SemaphoreType.DMA,
        ],
        compiler_params=pltpu.CompilerParams(use_tc_tiling_on_sc=False),
    )(idx, token_table, pos_table)


def kernel(inputs, token_table, pos_table):
    idx = inputs.astype(jnp.int32).reshape(
        NW, CHUNKS_PER_W * GATHERS_PER_CHUNK, GATHER_LEN)
    return _run(idx, token_table, pos_table)
